# baseline (speedup 1.0000x reference)
import functools

import jax
import jax.numpy as jnp
from jax import lax
from jax.experimental import pallas as pl
from jax.experimental.pallas import tpu as pltpu
from jax.experimental.pallas import tpu_sc as plsc

_NC = 2
_NS = 16
_NW = _NC * _NS


@functools.lru_cache(maxsize=None)
def _make_gather(V, D, B):
  b_per_w = B // _NW
  n_grp = b_per_w // 16
  mesh = plsc.VectorSubcoreMesh(core_axis_name="c", subcore_axis_name="s")

  @functools.partial(
      pl.kernel,
      mesh=mesh,
      out_type=jax.ShapeDtypeStruct((D, B), jnp.float32),
      scratch_types=[
          pltpu.VMEM((b_per_w,), jnp.int32),
          pltpu.VMEM((b_per_w, D), jnp.float32),
          pltpu.VMEM((D, b_per_w), jnp.float32),
          pltpu.SemaphoreType.DMA,
      ],
      compiler_params=pltpu.CompilerParams(needs_layout_passes=False),
  )
  def gather_kernel(table_hbm, idx_hbm, outT_hbm, idx_v, rows_v, rowsT_v, sem):
    wid = lax.axis_index("s") * _NC + lax.axis_index("c")
    base = wid * b_per_w
    pltpu.sync_copy(idx_hbm.at[pl.ds(base, b_per_w)], idx_v)
    lane = lax.iota(jnp.int32, 16)

    @plsc.parallel_loop(0, n_grp, unroll=2)
    def issue16(g):
      v = idx_v[pl.ds(g * 16, 16)]
      for k in range(16):
        r = jnp.max(jnp.where(lane == k, v, 0))
        pltpu.async_copy(
            table_hbm.at[pl.ds(r, 1)], rows_v.at[pl.ds(g * 16 + k, 1)], sem)

    pltpu.make_async_copy(
        table_hbm.at[pl.ds(0, b_per_w)], rows_v, sem).wait()

    @plsc.parallel_loop(0, n_grp, unroll=4)
    def tblock(g):
      jvec = g * 16 + lane
      for c in range(D):
        cvec = jnp.full((16,), c, jnp.int32)
        val = plsc.load_gather(rows_v, [jvec, cvec])
        rowsT_v[c, pl.ds(g * 16, 16)] = val
    pltpu.sync_copy(rowsT_v, outT_hbm.at[:, pl.ds(base, b_per_w)])

  return gather_kernel


def kernel(customer_id, user_embedding_table):
  (B,) = customer_id.shape
  V, D = user_embedding_table.shape
  outT = _make_gather(V, D, B)(user_embedding_table,
                               customer_id.astype(jnp.int32))
  return outT.T


# speedup vs baseline: 1.0230x; 1.0230x over previous
import functools

import jax
import jax.numpy as jnp
from jax import lax
from jax.experimental import pallas as pl
from jax.experimental.pallas import tpu as pltpu
from jax.experimental.pallas import tpu_sc as plsc

_NC = 2
_NS = 16
_NW = _NC * _NS


@functools.lru_cache(maxsize=None)
def _make_gather(V, D, B):
  b_per_w = B // _NW
  n_grp = b_per_w // 16
  mesh = plsc.VectorSubcoreMesh(core_axis_name="c", subcore_axis_name="s")

  @functools.partial(
      pl.kernel,
      mesh=mesh,
      out_type=jax.ShapeDtypeStruct((D, B), jnp.float32),
      scratch_types=[
          pltpu.VMEM((b_per_w,), jnp.int32),
          pltpu.VMEM((b_per_w, D), jnp.float32),
          pltpu.VMEM((D, b_per_w), jnp.float32),
          pltpu.SemaphoreType.DMA,
      ],
      compiler_params=pltpu.CompilerParams(needs_layout_passes=False),
  )
  def gather_kernel(table_hbm, idx_hbm, outT_hbm, idx_v, rows_v, rowsT_v, sem):
    wid = lax.axis_index("s") * _NC + lax.axis_index("c")
    base = wid * b_per_w
    pltpu.sync_copy(idx_hbm.at[pl.ds(base, b_per_w)], idx_v)
    lane = lax.iota(jnp.int32, 16)

    @plsc.parallel_loop(0, n_grp, unroll=2)
    def issue16(g):
      v = idx_v[pl.ds(g * 16, 16)]
      for k in range(16):
        r = jnp.max(jnp.where(lane == k, v, 0))
        pltpu.async_copy(
            table_hbm.at[pl.ds(r, 1)], rows_v.at[pl.ds(g * 16 + k, 1)], sem)

    pltpu.make_async_copy(
        table_hbm.at[pl.ds(0, b_per_w)], rows_v, sem).wait()

    @plsc.parallel_loop(0, n_grp, unroll=2)
    def tblock(g):
      jvec = g * 16 + lane
      for c in range(D):
        cvec = jnp.full((16,), c, jnp.int32)
        val = plsc.load_gather(rows_v, [jvec, cvec])
        rowsT_v[c, pl.ds(g * 16, 16)] = val
    pltpu.sync_copy(rowsT_v, outT_hbm.at[:, pl.ds(base, b_per_w)])

  return gather_kernel


def kernel(customer_id, user_embedding_table):
  (B,) = customer_id.shape
  V, D = user_embedding_table.shape
  outT = _make_gather(V, D, B)(user_embedding_table,
                               customer_id.astype(jnp.int32))
  return outT.T


# speedup vs baseline: 1.0370x; 1.0137x over previous
import functools

import jax
import jax.numpy as jnp
from jax import lax
from jax.experimental import pallas as pl
from jax.experimental.pallas import tpu as pltpu
from jax.experimental.pallas import tpu_sc as plsc

_NC = 2
_NS = 16
_NW = _NC * _NS


@functools.lru_cache(maxsize=None)
def _make_gather(V, D, B):
  b_per_w = B // _NW
  n_grp = b_per_w // 16
  mesh = plsc.VectorSubcoreMesh(core_axis_name="c", subcore_axis_name="s")

  @functools.partial(
      pl.kernel,
      mesh=mesh,
      out_type=jax.ShapeDtypeStruct((D, B), jnp.float32),
      scratch_types=[
          pltpu.VMEM((b_per_w,), jnp.int32),
          pltpu.VMEM((b_per_w, D), jnp.float32),
          pltpu.VMEM((D, b_per_w), jnp.float32),
          pltpu.SemaphoreType.DMA,
      ],
      compiler_params=pltpu.CompilerParams(needs_layout_passes=False),
  )
  def gather_kernel(table_hbm, idx_hbm, outT_hbm, idx_v, rows_v, rowsT_v, sem):
    wid = lax.axis_index("s") * _NC + lax.axis_index("c")
    base = wid * b_per_w
    pltpu.sync_copy(idx_hbm.at[pl.ds(base, b_per_w)], idx_v)
    lane = lax.iota(jnp.int32, 16)

    @plsc.parallel_loop(0, n_grp)
    def issue16(g):
      v = idx_v[pl.ds(g * 16, 16)]
      for k in range(16):
        r = jnp.max(jnp.where(lane == k, v, 0))
        pltpu.async_copy(
            table_hbm.at[pl.ds(r, 1)], rows_v.at[pl.ds(g * 16 + k, 1)], sem)

    pltpu.make_async_copy(
        table_hbm.at[pl.ds(0, b_per_w)], rows_v, sem).wait()

    @plsc.parallel_loop(0, n_grp)
    def tblock(g):
      jvec = g * 16 + lane
      for c in range(D):
        cvec = jnp.full((16,), c, jnp.int32)
        val = plsc.load_gather(rows_v, [jvec, cvec])
        rowsT_v[c, pl.ds(g * 16, 16)] = val
    pltpu.sync_copy(rowsT_v, outT_hbm.at[:, pl.ds(base, b_per_w)])

  return gather_kernel


def kernel(customer_id, user_embedding_table):
  (B,) = customer_id.shape
  V, D = user_embedding_table.shape
  outT = _make_gather(V, D, B)(user_embedding_table,
                               customer_id.astype(jnp.int32))
  return outT.T
